# speedup vs baseline: 8.9659x; 8.9659x over previous
import functools
import jax
import jax.numpy as jnp
from jax import lax
from jax.experimental import pallas as pl
from jax.experimental.pallas import tpu as pltpu

N_NODES = 10000
DIM_IN = 128
DIM_H = 64
ROW_BLK = 400


def _xw_y_kernel(x_ref, w_ref, deg_ref, y_ref, dinv_ref):
    xw = jnp.dot(x_ref[...], w_ref[...], preferred_element_type=jnp.float32)
    dinv = lax.rsqrt(deg_ref[...])
    y_ref[...] = xw * dinv
    dinv_ref[...] = dinv


def _compute_y(x, w_gcn, deg):
    nb = N_NODES // ROW_BLK
    return pl.pallas_call(
        _xw_y_kernel,
        grid=(nb,),
        in_specs=[
            pl.BlockSpec((ROW_BLK, DIM_IN), lambda i: (i, 0)),
            pl.BlockSpec((DIM_IN, DIM_H), lambda i: (0, 0)),
            pl.BlockSpec((ROW_BLK, 1), lambda i: (i, 0)),
        ],
        out_specs=[
            pl.BlockSpec((ROW_BLK, DIM_H), lambda i: (i, 0)),
            pl.BlockSpec((ROW_BLK, 1), lambda i: (i, 0)),
        ],
        out_shape=[
            jax.ShapeDtypeStruct((N_NODES, DIM_H), jnp.float32),
            jax.ShapeDtypeStruct((N_NODES, 1), jnp.float32),
        ],
    )(x, w_gcn, deg)


def _lstm_kernel(agg_ref, dinv_ref, bg_ref,
                 wii_ref, wif_ref, wig_ref, wio_ref,
                 bi_ref, bf_ref, bgg_ref, bo_ref,
                 whi_ref, whf_ref, whg_ref, who_ref,
                 out_ref,
                 gi_s, gf_s, gg_s, go_s, h_s, c_s):
    step = pl.program_id(0)

    @pl.when(step == 0)
    def _():
        h_s[...] = jnp.zeros_like(h_s)
        c_s[...] = jnp.zeros_like(c_s)

    hg = jnp.maximum(dinv_ref[...] * agg_ref[...] + bg_ref[...], 0.0)
    gi_s[...] = jnp.dot(hg, wii_ref[...], preferred_element_type=jnp.float32) + bi_ref[...]
    gf_s[...] = jnp.dot(hg, wif_ref[...], preferred_element_type=jnp.float32) + bf_ref[...]
    gg_s[...] = jnp.dot(hg, wig_ref[...], preferred_element_type=jnp.float32) + bgg_ref[...]
    go_s[...] = jnp.dot(hg, wio_ref[...], preferred_element_type=jnp.float32) + bo_ref[...]

    whi = whi_ref[...]
    whf = whf_ref[...]
    whg = whg_ref[...]
    who = who_ref[...]

    def body(r, carry):
        h, c = carry
        ig = jax.nn.sigmoid(gi_s[pl.ds(r, 1), :]
                            + jnp.dot(h, whi, preferred_element_type=jnp.float32))
        fg = jax.nn.sigmoid(gf_s[pl.ds(r, 1), :]
                            + jnp.dot(h, whf, preferred_element_type=jnp.float32))
        gg = jnp.tanh(gg_s[pl.ds(r, 1), :]
                      + jnp.dot(h, whg, preferred_element_type=jnp.float32))
        og = jax.nn.sigmoid(go_s[pl.ds(r, 1), :]
                            + jnp.dot(h, who, preferred_element_type=jnp.float32))
        c_new = fg * c + ig * gg
        h_new = og * jnp.tanh(c_new)
        out_ref[pl.ds(r, 1), :] = h_new
        return h_new, c_new

    h_fin, c_fin = lax.fori_loop(0, ROW_BLK, body, (h_s[...], c_s[...]))
    h_s[...] = h_fin
    c_s[...] = c_fin


def _lstm_stage(agg, dinv, b_gcn, wih_t, whh_t, bias):
    nb = N_NODES // ROW_BLK
    wih = [wih_t[:, k * DIM_H:(k + 1) * DIM_H] for k in range(4)]
    whh = [whh_t[:, k * DIM_H:(k + 1) * DIM_H] for k in range(4)]
    bs = [bias[None, k * DIM_H:(k + 1) * DIM_H] for k in range(4)]
    full = lambda i: (0, 0)
    blk = lambda i: (i, 0)
    return pl.pallas_call(
        _lstm_kernel,
        grid=(nb,),
        in_specs=[
            pl.BlockSpec((ROW_BLK, DIM_H), blk),
            pl.BlockSpec((ROW_BLK, 1), blk),
            pl.BlockSpec((1, DIM_H), full),
            *[pl.BlockSpec((DIM_H, DIM_H), full) for _ in range(4)],
            *[pl.BlockSpec((1, DIM_H), full) for _ in range(4)],
            *[pl.BlockSpec((DIM_H, DIM_H), full) for _ in range(4)],
        ],
        out_specs=pl.BlockSpec((ROW_BLK, DIM_H), blk),
        out_shape=jax.ShapeDtypeStruct((N_NODES, DIM_H), jnp.float32),
        scratch_shapes=[
            pltpu.VMEM((ROW_BLK, DIM_H), jnp.float32) for _ in range(4)
        ] + [
            pltpu.VMEM((1, DIM_H), jnp.float32),
            pltpu.VMEM((1, DIM_H), jnp.float32),
        ],
    )(agg, dinv, b_gcn[None, :], *wih, *bs, *whh)


def kernel(x, edge_index, W_gcn, b_gcn, W_ih, W_hh, b_ih, b_hh):
    src = edge_index[0]
    dst = edge_index[1]
    deg = 1.0 + jax.ops.segment_sum(
        jnp.ones((src.shape[0],), jnp.float32), dst, num_segments=N_NODES)
    y, dinv = _compute_y(x, W_gcn, deg[:, None])
    agg = y + jax.ops.segment_sum(jnp.take(y, src, axis=0), dst,
                                  num_segments=N_NODES)
    bias = b_ih + b_hh
    return _lstm_stage(agg, dinv, b_gcn, W_ih.T, W_hh.T, bias)


# speedup vs baseline: 16.6796x; 1.8603x over previous
import functools
import jax
import jax.numpy as jnp
from jax import lax
from jax.experimental import pallas as pl
from jax.experimental.pallas import tpu as pltpu
from jax.experimental.pallas import tpu_sc as plsc

N_NODES = 10000
N_PAD = 10240
DIM_IN = 128
DIM_H = 64
N_EDGES = 320000
ROW_BLK = 400

NC = 2
NS = 16
NW = NC * NS
EDGES_PER_TILE = N_EDGES // NW
CHUNK = 80
N_CHUNKS = EDGES_PER_TILE // CHUNK
ROWS_PER_TILE = N_PAD // NS

_sc_mesh = plsc.VectorSubcoreMesh(
    core_axis_name="c", subcore_axis_name="s", num_cores=NC, num_subcores=NS)
_sc_params = pltpu.CompilerParams(use_tc_tiling_on_sc=False)


@functools.partial(
    pl.kernel,
    out_type=jax.ShapeDtypeStruct((NC, N_PAD, 16), jnp.float32),
    mesh=_sc_mesh,
    compiler_params=_sc_params,
    scratch_types=[
        pltpu.VMEM((CHUNK,), jnp.int32),
        pltpu.VMEM((CHUNK, 16), jnp.float32),
        pltpu.VMEM_SHARED((N_PAD, 16), jnp.float32),
    ],
)
def _sc_degree(dst_hbm, zeros_hbm, ones_hbm, out_hbm, idx_v, ones_v, cnt_sh):
    c = lax.axis_index("c")
    s = lax.axis_index("s")
    w = c * NS + s
    row0 = s * ROWS_PER_TILE
    pltpu.sync_copy(zeros_hbm.at[pl.ds(row0, ROWS_PER_TILE)],
                    cnt_sh.at[pl.ds(row0, ROWS_PER_TILE)])
    pltpu.sync_copy(ones_hbm, ones_v)
    plsc.subcore_barrier()
    base = w * EDGES_PER_TILE

    def chunk(i, carry):
        pltpu.sync_copy(dst_hbm.at[pl.ds(base + i * CHUNK, CHUNK)], idx_v)
        pltpu.sync_copy(ones_v, cnt_sh.at[idx_v], add=True)
        return carry

    lax.fori_loop(0, N_CHUNKS, chunk, 0)
    plsc.subcore_barrier()
    pltpu.sync_copy(cnt_sh.at[pl.ds(row0, ROWS_PER_TILE)],
                    out_hbm.at[c, pl.ds(row0, ROWS_PER_TILE)])


@functools.partial(
    pl.kernel,
    out_type=jax.ShapeDtypeStruct((NC, N_PAD, DIM_H), jnp.float32),
    mesh=_sc_mesh,
    compiler_params=_sc_params,
    scratch_types=[
        pltpu.VMEM((CHUNK,), jnp.int32),
        pltpu.VMEM((CHUNK,), jnp.int32),
        pltpu.VMEM((CHUNK, DIM_H), jnp.float32),
        pltpu.VMEM_SHARED((N_PAD, DIM_H), jnp.float32),
        pltpu.SemaphoreType.DMA,
    ],
)
def _sc_aggregate(src_hbm, dst_hbm, y_hbm, out_hbm,
                  isrc_v, idst_v, rows_v, agg_sh, sem):
    c = lax.axis_index("c")
    s = lax.axis_index("s")
    w = c * NS + s
    row0 = s * ROWS_PER_TILE
    pltpu.sync_copy(y_hbm.at[pl.ds(row0, ROWS_PER_TILE)],
                    agg_sh.at[pl.ds(row0, ROWS_PER_TILE)])
    plsc.subcore_barrier()
    base = w * EDGES_PER_TILE

    def chunk(i, carry):
        pltpu.sync_copy(src_hbm.at[pl.ds(base + i * CHUNK, CHUNK)], isrc_v)
        pltpu.sync_copy(dst_hbm.at[pl.ds(base + i * CHUNK, CHUNK)], idst_v)
        pltpu.async_copy(y_hbm.at[isrc_v], rows_v, sem).wait()
        pltpu.sync_copy(rows_v, agg_sh.at[idst_v], add=True)
        return carry

    lax.fori_loop(0, N_CHUNKS, chunk, 0)
    plsc.subcore_barrier()
    pltpu.sync_copy(agg_sh.at[pl.ds(row0, ROWS_PER_TILE)],
                    out_hbm.at[c, pl.ds(row0, ROWS_PER_TILE)])


def _xw_y_kernel(x_ref, w_ref, cnt_ref, y_ref, dinv_ref):
    xw = jnp.dot(x_ref[...], w_ref[...], preferred_element_type=jnp.float32)
    deg = 1.0 + cnt_ref[0, :, 0:1] + cnt_ref[1, :, 0:1]
    dinv = lax.rsqrt(deg)
    y_ref[...] = xw * dinv
    dinv_ref[...] = dinv


def _compute_y(x, w_gcn, cnt):
    nb = N_NODES // ROW_BLK
    return pl.pallas_call(
        _xw_y_kernel,
        grid=(nb,),
        in_specs=[
            pl.BlockSpec((ROW_BLK, DIM_IN), lambda i: (i, 0)),
            pl.BlockSpec((DIM_IN, DIM_H), lambda i: (0, 0)),
            pl.BlockSpec((2, ROW_BLK, 16), lambda i: (0, i, 0)),
        ],
        out_specs=[
            pl.BlockSpec((ROW_BLK, DIM_H), lambda i: (i, 0)),
            pl.BlockSpec((ROW_BLK, 1), lambda i: (i, 0)),
        ],
        out_shape=[
            jax.ShapeDtypeStruct((N_PAD, DIM_H), jnp.float32),
            jax.ShapeDtypeStruct((N_NODES, 1), jnp.float32),
        ],
    )(x, w_gcn, cnt)


def _lstm_kernel(agg_ref, y_ref, dinv_ref, bg_ref,
                 wii_ref, wif_ref, wig_ref, wio_ref,
                 bi_ref, bf_ref, bgg_ref, bo_ref,
                 whi_ref, whf_ref, whg_ref, who_ref,
                 out_ref,
                 gi_s, gf_s, gg_s, go_s, h_s, c_s):
    step = pl.program_id(0)

    @pl.when(step == 0)
    def _():
        h_s[...] = jnp.zeros_like(h_s)
        c_s[...] = jnp.zeros_like(c_s)

    agg = agg_ref[0] + agg_ref[1] - y_ref[...]
    hg = jnp.maximum(dinv_ref[...] * agg + bg_ref[...], 0.0)
    gi_s[...] = jnp.dot(hg, wii_ref[...], preferred_element_type=jnp.float32) + bi_ref[...]
    gf_s[...] = jnp.dot(hg, wif_ref[...], preferred_element_type=jnp.float32) + bf_ref[...]
    gg_s[...] = jnp.dot(hg, wig_ref[...], preferred_element_type=jnp.float32) + bgg_ref[...]
    go_s[...] = jnp.dot(hg, wio_ref[...], preferred_element_type=jnp.float32) + bo_ref[...]

    whi = whi_ref[...]
    whf = whf_ref[...]
    whg = whg_ref[...]
    who = who_ref[...]

    def body(r, carry):
        h, c = carry
        ig = jax.nn.sigmoid(gi_s[pl.ds(r, 1), :]
                            + jnp.dot(h, whi, preferred_element_type=jnp.float32))
        fg = jax.nn.sigmoid(gf_s[pl.ds(r, 1), :]
                            + jnp.dot(h, whf, preferred_element_type=jnp.float32))
        gg = jnp.tanh(gg_s[pl.ds(r, 1), :]
                      + jnp.dot(h, whg, preferred_element_type=jnp.float32))
        og = jax.nn.sigmoid(go_s[pl.ds(r, 1), :]
                            + jnp.dot(h, who, preferred_element_type=jnp.float32))
        c_new = fg * c + ig * gg
        h_new = og * jnp.tanh(c_new)
        out_ref[pl.ds(r, 1), :] = h_new
        return h_new, c_new

    h_fin, c_fin = lax.fori_loop(0, ROW_BLK, body, (h_s[...], c_s[...]))
    h_s[...] = h_fin
    c_s[...] = c_fin


def _lstm_stage(agg, y, dinv, b_gcn, wih_t, whh_t, bias):
    nb = N_NODES // ROW_BLK
    wih = [wih_t[:, k * DIM_H:(k + 1) * DIM_H] for k in range(4)]
    whh = [whh_t[:, k * DIM_H:(k + 1) * DIM_H] for k in range(4)]
    bs = [bias[None, k * DIM_H:(k + 1) * DIM_H] for k in range(4)]
    full = lambda i: (0, 0)
    blk = lambda i: (i, 0)
    return pl.pallas_call(
        _lstm_kernel,
        grid=(nb,),
        in_specs=[
            pl.BlockSpec((2, ROW_BLK, DIM_H), lambda i: (0, i, 0)),
            pl.BlockSpec((ROW_BLK, DIM_H), blk),
            pl.BlockSpec((ROW_BLK, 1), blk),
            pl.BlockSpec((1, DIM_H), full),
            *[pl.BlockSpec((DIM_H, DIM_H), full) for _ in range(4)],
            *[pl.BlockSpec((1, DIM_H), full) for _ in range(4)],
            *[pl.BlockSpec((DIM_H, DIM_H), full) for _ in range(4)],
        ],
        out_specs=pl.BlockSpec((ROW_BLK, DIM_H), blk),
        out_shape=jax.ShapeDtypeStruct((N_NODES, DIM_H), jnp.float32),
        scratch_shapes=[
            pltpu.VMEM((ROW_BLK, DIM_H), jnp.float32) for _ in range(4)
        ] + [
            pltpu.VMEM((1, DIM_H), jnp.float32),
            pltpu.VMEM((1, DIM_H), jnp.float32),
        ],
    )(agg, y, dinv, b_gcn[None, :], *wih, *bs, *whh)


def kernel(x, edge_index, W_gcn, b_gcn, W_ih, W_hh, b_ih, b_hh):
    src = edge_index[0]
    dst = edge_index[1]
    zeros16 = jnp.zeros((N_PAD, 16), jnp.float32)
    ones16 = jnp.ones((CHUNK, 16), jnp.float32)
    cnt = _sc_degree(dst, zeros16, ones16)
    y, dinv = _compute_y(x, W_gcn, cnt)
    agg = _sc_aggregate(src, dst, y)
    bias = b_ih + b_hh
    return _lstm_stage(agg, y, dinv, b_gcn, W_ih.T, W_hh.T, bias)
